# Initial kernel scaffold; baseline (speedup 1.0000x reference)
import jax
import jax.numpy as jnp
from jax.experimental import pallas as pl


def kernel(x, edge_index, W1, b1, W2, b2):
    raise NotImplementedError("write your pallas kernel here")


# speedup vs baseline: 7.4680x; 7.4680x over previous
import functools

import jax
import jax.numpy as jnp
from jax import lax
from jax.experimental import pallas as pl
from jax.experimental.pallas import tpu as pltpu
from jax.experimental.pallas import tpu_sc as plsc

NC = 2
NS = 16
CH = 128


def _make_agg(n_pad, e_pad, h):
  per_tile = e_pad // NS
  n_chunks = per_tile // CH
  per_drain = n_pad // NS
  mesh = plsc.VectorSubcoreMesh(
      core_axis_name="c", subcore_axis_name="s", num_cores=NC, num_subcores=NS)

  @functools.partial(
      pl.kernel,
      out_type=jax.ShapeDtypeStruct((2 * h, 128), jnp.float32),
      mesh=mesh,
      scratch_types=[
          pltpu.VMEM((CH,), jnp.int32),
          pltpu.VMEM((CH,), jnp.int32),
          pltpu.VMEM((CH, 128), jnp.float32),
          pltpu.VMEM_SHARED((n_pad, 128), jnp.float32),
          pltpu.SemaphoreType.DMA,
      ],
  )
  def agg(ycat, src2, dst, zeros, out, sidx_v, didx_v, rows_v, acc, sem):
    c = lax.axis_index("c")
    s = lax.axis_index("s")
    pltpu.sync_copy(zeros, acc.at[pl.ds(s * per_drain, per_drain)])
    plsc.subcore_barrier()

    def body(i, carry):
      base = s * per_tile + i * CH
      pltpu.sync_copy(src2.at[pl.ds(c * e_pad + base, CH)], sidx_v)
      pltpu.sync_copy(dst.at[pl.ds(base, CH)], didx_v)
      pltpu.async_copy(ycat.at[sidx_v], rows_v, sem).wait()
      pltpu.sync_copy(rows_v, acc.at[didx_v], add=True)
      return carry

    lax.fori_loop(0, n_chunks, body, 0)
    plsc.subcore_barrier()
    pltpu.sync_copy(acc.at[pl.ds(s * per_drain, per_drain)],
                    out.at[pl.ds(c * h + s * per_drain, per_drain)])

  return agg


def _make_deg(n_pad, e_pad, h):
  per_core = e_pad // NC
  per_tile = per_core // NS
  n_chunks = per_tile // CH
  per_drain = n_pad // NS
  mesh = plsc.VectorSubcoreMesh(
      core_axis_name="c", subcore_axis_name="s", num_cores=NC, num_subcores=NS)

  @functools.partial(
      pl.kernel,
      out_type=jax.ShapeDtypeStruct((2 * h, 128), jnp.float32),
      mesh=mesh,
      scratch_types=[
          pltpu.VMEM((CH,), jnp.int32),
          pltpu.VMEM((CH, 128), jnp.float32),
          pltpu.VMEM_SHARED((n_pad, 128), jnp.float32),
      ],
  )
  def deg(dst, zeros, ones, out, didx_v, ones_v, acc):
    c = lax.axis_index("c")
    s = lax.axis_index("s")
    pltpu.sync_copy(ones, ones_v)
    pltpu.sync_copy(zeros, acc.at[pl.ds(s * per_drain, per_drain)])
    plsc.subcore_barrier()

    def body(i, carry):
      base = c * per_core + s * per_tile + i * CH
      pltpu.sync_copy(dst.at[pl.ds(base, CH)], didx_v)
      pltpu.sync_copy(ones_v, acc.at[didx_v], add=True)
      return carry

    lax.fori_loop(0, n_chunks, body, 0)
    plsc.subcore_barrier()
    pltpu.sync_copy(acc.at[pl.ds(s * per_drain, per_drain)],
                    out.at[pl.ds(c * h + s * per_drain, per_drain)])

  return deg


def _dinv(d_ref):
  return lax.rsqrt(d_ref[0, :, 0] + d_ref[1, :, 0] + 1.0)[:, None]


def _prep(x, dg3, n, r, h):
  def body(x_ref, d_ref, y_ref):
    y = x_ref[...] * _dinv(d_ref)
    y_ref[0] = y[:, :128]
    y_ref[1] = y[:, 128:]

  return pl.pallas_call(
      body,
      grid=(n // r,),
      in_specs=[pl.BlockSpec((r, 256), lambda i: (i, 0)),
                pl.BlockSpec((2, r, 128), lambda i: (0, i, 0))],
      out_specs=pl.BlockSpec((2, r, 128), lambda i: (0, i, 0)),
      out_shape=jax.ShapeDtypeStruct((2, h, 128), jnp.float32),
  )(x, dg3)


def _mid(agg3, ytab3, dg3, W1, b1, W2, n, r, h):
  hid = W1.shape[1]
  inc = W1.shape[0]

  def body(a_ref, y_ref, d_ref, w1_ref, b1_ref, w2_ref, o_ref):
    dinv = _dinv(d_ref)
    u = jnp.concatenate([a_ref[0] + y_ref[0], a_ref[1] + y_ref[1]],
                        axis=1) * dinv
    t = jnp.dot(u, w1_ref[...], preferred_element_type=jnp.float32)
    t = jnp.maximum(t + b1_ref[...], 0.0)
    y2 = jnp.dot(t, w2_ref[...], preferred_element_type=jnp.float32) * dinv
    o_ref[0] = y2[:, :128]
    o_ref[1] = y2[:, 128:]

  return pl.pallas_call(
      body,
      grid=(n // r,),
      in_specs=[pl.BlockSpec((2, r, 128), lambda i: (0, i, 0)),
                pl.BlockSpec((2, r, 128), lambda i: (0, i, 0)),
                pl.BlockSpec((2, r, 128), lambda i: (0, i, 0)),
                pl.BlockSpec((inc, hid), lambda i: (0, 0)),
                pl.BlockSpec((hid,), lambda i: (0,)),
                pl.BlockSpec((hid, 256), lambda i: (0, 0))],
      out_specs=pl.BlockSpec((2, r, 128), lambda i: (0, i, 0)),
      out_shape=jax.ShapeDtypeStruct((2, h, 128), jnp.float32),
  )(agg3, ytab3, dg3, W1, b1, W2)


def _final(aggc3, y2tab3, dg3, b2, n, r):
  outc = b2.shape[0]

  def body(a_ref, y_ref, d_ref, b2_ref, o_ref):
    o = jnp.concatenate([a_ref[0] + y_ref[0], a_ref[1] + y_ref[1]],
                        axis=1) * _dinv(d_ref)
    o_ref[...] = jnp.tanh(o + b2_ref[...])

  return pl.pallas_call(
      body,
      grid=(n // r,),
      in_specs=[pl.BlockSpec((2, r, 128), lambda i: (0, i, 0)),
                pl.BlockSpec((2, r, 128), lambda i: (0, i, 0)),
                pl.BlockSpec((2, r, 128), lambda i: (0, i, 0)),
                pl.BlockSpec((outc,), lambda i: (0,))],
      out_specs=pl.BlockSpec((r, outc), lambda i: (i, 0)),
      out_shape=jax.ShapeDtypeStruct((n, outc), jnp.float32),
  )(aggc3, y2tab3, dg3, b2)


def kernel(x, edge_index, W1, b1, W2, b2):
  n = x.shape[0]
  e = edge_index.shape[1]
  quantum = NC * NS * CH
  e_pad = ((e + quantum - 1) // quantum) * quantum
  n_quant = NS * 8
  n_pad = ((n + n_quant - 1) // n_quant) * n_quant
  r = 1000 if n % 1000 == 0 else n // 8
  h = ((n_pad + r - 1) // r) * r
  trash = n

  src = edge_index[0].astype(jnp.int32)
  dst = edge_index[1].astype(jnp.int32)
  pad = e_pad - e
  src_p = jnp.concatenate([src, jnp.zeros((pad,), jnp.int32)])
  dst_p = jnp.concatenate([dst, jnp.full((pad,), trash, jnp.int32)])
  src2 = jnp.concatenate([src_p, src_p + h])
  zeros128 = jnp.zeros((n_pad // NS, 128), jnp.float32)
  ones128 = jnp.ones((CH, 128), jnp.float32)

  deg_k = _make_deg(n_pad, e_pad, h)
  agg_k = _make_agg(n_pad, e_pad, h)

  dg3 = deg_k(dst_p, zeros128, ones128).reshape(2, h, 128)
  ytab3 = _prep(x, dg3, n, r, h)
  agg3 = agg_k(ytab3.reshape(2 * h, 128), src2, dst_p,
               zeros128).reshape(2, h, 128)
  y2tab3 = _mid(agg3, ytab3, dg3, W1, b1, W2, n, r, h)
  aggc3 = agg_k(y2tab3.reshape(2 * h, 128), src2, dst_p,
                zeros128).reshape(2, h, 128)
  return _final(aggc3, y2tab3, dg3, b2, n, r)


# speedup vs baseline: 9.5529x; 1.2792x over previous
import functools

import jax
import jax.numpy as jnp
from jax import lax
from jax.experimental import pallas as pl
from jax.experimental.pallas import tpu as pltpu
from jax.experimental.pallas import tpu_sc as plsc

NC = 2
NS = 16
CH = 128


def _make_agg(n_pad, e_pad, h):
  per_tile = e_pad // NS
  n_chunks = per_tile // CH
  nbuf = 2
  nph = 2
  ph_chunks = n_chunks // nph
  per_drain = n_pad // NS
  mesh = plsc.VectorSubcoreMesh(
      core_axis_name="c", subcore_axis_name="s", num_cores=NC, num_subcores=NS)

  @functools.partial(
      pl.kernel,
      out_type=jax.ShapeDtypeStruct((2 * h, 128), jnp.float32),
      mesh=mesh,
      scratch_types=[
          pltpu.VMEM((ph_chunks, CH), jnp.int32),
          pltpu.VMEM((ph_chunks, CH), jnp.int32),
          pltpu.VMEM((nbuf, CH, 128), jnp.float32),
          pltpu.VMEM_SHARED((n_pad, 128), jnp.float32),
          pltpu.SemaphoreType.DMA,
      ],
  )
  def agg(ycat, src3, dst3, zeros, out, sidx_all, didx_all, rows, acc, gsem):
    c = lax.axis_index("c")
    s = lax.axis_index("s")
    pltpu.sync_copy(zeros, acc.at[pl.ds(s * per_drain, per_drain)])
    plsc.subcore_barrier()

    def body(i, carry):
      gcps = [pltpu.async_copy(ycat.at[sidx_all.at[i * nbuf + b]],
                               rows.at[b], gsem) for b in range(nbuf)]
      for b in range(nbuf):
        gcps[b].wait()
        pltpu.sync_copy(rows.at[b], acc.at[didx_all.at[i * nbuf + b]],
                        add=True)
      return carry

    for ph in range(nph):
      pltpu.sync_copy(src3.at[c * NS + s, pl.ds(ph * ph_chunks, ph_chunks)],
                      sidx_all)
      pltpu.sync_copy(dst3.at[s, pl.ds(ph * ph_chunks, ph_chunks)], didx_all)
      lax.fori_loop(0, ph_chunks // nbuf, body, 0)
    plsc.subcore_barrier()
    pltpu.sync_copy(acc.at[pl.ds(s * per_drain, per_drain)],
                    out.at[pl.ds(c * h + s * per_drain, per_drain)])

  return agg


def _make_deg(n_pad, e_pad, h):
  per_core = e_pad // NC
  per_tile = per_core // NS
  n_chunks = per_tile // CH
  nbuf = 1
  n_outer = n_chunks // nbuf
  per_drain = n_pad // NS
  mesh = plsc.VectorSubcoreMesh(
      core_axis_name="c", subcore_axis_name="s", num_cores=NC, num_subcores=NS)

  @functools.partial(
      pl.kernel,
      out_type=jax.ShapeDtypeStruct((2 * h, 128), jnp.float32),
      mesh=mesh,
      scratch_types=[
          pltpu.VMEM((n_chunks, CH), jnp.int32),
          pltpu.VMEM((CH, 128), jnp.float32),
          pltpu.VMEM_SHARED((n_pad, 128), jnp.float32),
          pltpu.SemaphoreType.DMA,
      ],
  )
  def deg(dst3, zeros, ones, out, didx_all, ones_v, acc, ssem):
    c = lax.axis_index("c")
    s = lax.axis_index("s")
    pltpu.sync_copy(ones, ones_v)
    pltpu.sync_copy(dst3.at[c * NS + s], didx_all)
    pltpu.sync_copy(zeros, acc.at[pl.ds(s * per_drain, per_drain)])
    plsc.subcore_barrier()

    def body(i, carry):
      pltpu.sync_copy(ones_v, acc.at[didx_all.at[i]], add=True)
      return carry

    lax.fori_loop(0, n_outer, body, 0)
    plsc.subcore_barrier()
    pltpu.sync_copy(acc.at[pl.ds(s * per_drain, per_drain)],
                    out.at[pl.ds(c * h + s * per_drain, per_drain)])

  return deg


def _dinv(d_ref):
  return lax.rsqrt(d_ref[0, :, 0] + d_ref[1, :, 0] + 1.0)[:, None]


def _prep(x, dg3, n, r, h):
  def body(x_ref, d_ref, y_ref):
    y = x_ref[...] * _dinv(d_ref)
    y_ref[0] = y[:, :128]
    y_ref[1] = y[:, 128:]

  return pl.pallas_call(
      body,
      grid=(n // r,),
      in_specs=[pl.BlockSpec((r, 256), lambda i: (i, 0)),
                pl.BlockSpec((2, r, 128), lambda i: (0, i, 0))],
      out_specs=pl.BlockSpec((2, r, 128), lambda i: (0, i, 0)),
      out_shape=jax.ShapeDtypeStruct((2, h, 128), jnp.float32),
  )(x, dg3)


def _mid(agg3, ytab3, dg3, W1, b1, W2, n, r, h):
  hid = W1.shape[1]
  inc = W1.shape[0]

  def body(a_ref, y_ref, d_ref, w1_ref, b1_ref, w2_ref, o_ref):
    dinv = _dinv(d_ref)
    u = jnp.concatenate([a_ref[0] + y_ref[0], a_ref[1] + y_ref[1]],
                        axis=1) * dinv
    t = jnp.dot(u, w1_ref[...], preferred_element_type=jnp.float32)
    t = jnp.maximum(t + b1_ref[...], 0.0)
    y2 = jnp.dot(t, w2_ref[...], preferred_element_type=jnp.float32) * dinv
    o_ref[0] = y2[:, :128]
    o_ref[1] = y2[:, 128:]

  return pl.pallas_call(
      body,
      grid=(n // r,),
      in_specs=[pl.BlockSpec((2, r, 128), lambda i: (0, i, 0)),
                pl.BlockSpec((2, r, 128), lambda i: (0, i, 0)),
                pl.BlockSpec((2, r, 128), lambda i: (0, i, 0)),
                pl.BlockSpec((inc, hid), lambda i: (0, 0)),
                pl.BlockSpec((hid,), lambda i: (0,)),
                pl.BlockSpec((hid, 256), lambda i: (0, 0))],
      out_specs=pl.BlockSpec((2, r, 128), lambda i: (0, i, 0)),
      out_shape=jax.ShapeDtypeStruct((2, h, 128), jnp.float32),
  )(agg3, ytab3, dg3, W1, b1, W2)


def _final(aggc3, y2tab3, dg3, b2, n, r):
  outc = b2.shape[0]

  def body(a_ref, y_ref, d_ref, b2_ref, o_ref):
    o = jnp.concatenate([a_ref[0] + y_ref[0], a_ref[1] + y_ref[1]],
                        axis=1) * _dinv(d_ref)
    o_ref[...] = jnp.tanh(o + b2_ref[...])

  return pl.pallas_call(
      body,
      grid=(n // r,),
      in_specs=[pl.BlockSpec((2, r, 128), lambda i: (0, i, 0)),
                pl.BlockSpec((2, r, 128), lambda i: (0, i, 0)),
                pl.BlockSpec((2, r, 128), lambda i: (0, i, 0)),
                pl.BlockSpec((outc,), lambda i: (0,))],
      out_specs=pl.BlockSpec((r, outc), lambda i: (i, 0)),
      out_shape=jax.ShapeDtypeStruct((n, outc), jnp.float32),
  )(aggc3, y2tab3, dg3, b2)


def kernel(x, edge_index, W1, b1, W2, b2):
  n = x.shape[0]
  e = edge_index.shape[1]
  quantum = NC * NS * CH
  e_pad = ((e + quantum - 1) // quantum) * quantum
  n_quant = NS * 8
  n_pad = ((n + n_quant - 1) // n_quant) * n_quant
  r = 1000 if n % 1000 == 0 else n // 8
  h = ((n_pad + r - 1) // r) * r
  trash = n

  src = edge_index[0].astype(jnp.int32)
  dst = edge_index[1].astype(jnp.int32)
  pad = e_pad - e
  src_p = jnp.concatenate([src, jnp.zeros((pad,), jnp.int32)])
  dst_p = jnp.concatenate([dst, jnp.full((pad,), trash, jnp.int32)])
  src3 = jnp.concatenate([src_p, src_p + h]).reshape(NC * NS, -1, CH)
  dst3_agg = dst_p.reshape(NS, -1, CH)
  dst3_deg = dst_p.reshape(NC * NS, -1, CH)
  zeros128 = jnp.zeros((n_pad // NS, 128), jnp.float32)
  ones128 = jnp.ones((CH, 128), jnp.float32)

  deg_k = _make_deg(n_pad, e_pad, h)
  agg_k = _make_agg(n_pad, e_pad, h)

  dg3 = deg_k(dst3_deg, zeros128, ones128).reshape(2, h, 128)
  ytab3 = _prep(x, dg3, n, r, h)
  agg3 = agg_k(ytab3.reshape(2 * h, 128), src3, dst3_agg,
               zeros128).reshape(2, h, 128)
  y2tab3 = _mid(agg3, ytab3, dg3, W1, b1, W2, n, r, h)
  aggc3 = agg_k(y2tab3.reshape(2 * h, 128), src3, dst3_agg,
                zeros128).reshape(2, h, 128)
  return _final(aggc3, y2tab3, dg3, b2, n, r)


# speedup vs baseline: 9.8911x; 1.0354x over previous
import functools

import jax
import jax.numpy as jnp
from jax import lax
from jax.experimental import pallas as pl
from jax.experimental.pallas import tpu as pltpu
from jax.experimental.pallas import tpu_sc as plsc

NC = 2
NS = 16
CH = 128


def _make_agg(n_pad, e_pad, h):
  per_tile = e_pad // NS
  n_chunks = per_tile // CH
  nbuf = 2
  nph = 2
  ph_chunks = n_chunks // nph
  per_drain = n_pad // NS
  mesh = plsc.VectorSubcoreMesh(
      core_axis_name="c", subcore_axis_name="s", num_cores=NC, num_subcores=NS)

  @functools.partial(
      pl.kernel,
      out_type=jax.ShapeDtypeStruct((2 * h, 128), jnp.float32),
      mesh=mesh,
      scratch_types=[
          pltpu.VMEM((ph_chunks, CH), jnp.int32),
          pltpu.VMEM((ph_chunks, CH), jnp.int32),
          pltpu.VMEM((nbuf, CH, 128), jnp.float32),
          pltpu.VMEM_SHARED((n_pad, 128), jnp.float32),
          pltpu.SemaphoreType.DMA,
          pltpu.SemaphoreType.DMA,
      ],
  )
  def agg(ycat, src3, dst3, zeros, out, sidx_all, didx_all, rows, acc, gsem,
          ssem):
    c = lax.axis_index("c")
    s = lax.axis_index("s")
    pltpu.sync_copy(zeros, acc.at[pl.ds(s * per_drain, per_drain)])
    plsc.subcore_barrier()

    def body(i, carry):
      gcps = [pltpu.async_copy(ycat.at[sidx_all.at[i * nbuf + b]],
                               rows.at[b], gsem) for b in range(nbuf)]
      scps = []
      for b in range(nbuf):
        gcps[b].wait()
        scps.append(pltpu.async_copy(
            rows.at[b], acc.at[didx_all.at[i * nbuf + b]], ssem, add=True))
      for cp in scps:
        cp.wait()
      return carry

    for ph in range(nph):
      pltpu.sync_copy(src3.at[c * NS + s, pl.ds(ph * ph_chunks, ph_chunks)],
                      sidx_all)
      pltpu.sync_copy(dst3.at[s, pl.ds(ph * ph_chunks, ph_chunks)], didx_all)
      lax.fori_loop(0, ph_chunks // nbuf, body, 0)
    plsc.subcore_barrier()
    pltpu.sync_copy(acc.at[pl.ds(s * per_drain, per_drain)],
                    out.at[pl.ds(c * h + s * per_drain, per_drain)])

  return agg


def _make_deg(n_pad, e_pad, h):
  per_core = e_pad // NC
  per_tile = per_core // NS
  n_chunks = per_tile // CH
  nbuf = 1
  n_outer = n_chunks // nbuf
  per_drain = n_pad // NS
  mesh = plsc.VectorSubcoreMesh(
      core_axis_name="c", subcore_axis_name="s", num_cores=NC, num_subcores=NS)

  @functools.partial(
      pl.kernel,
      out_type=jax.ShapeDtypeStruct((2 * h, 128), jnp.float32),
      mesh=mesh,
      scratch_types=[
          pltpu.VMEM((n_chunks, CH), jnp.int32),
          pltpu.VMEM((CH, 128), jnp.float32),
          pltpu.VMEM_SHARED((n_pad, 128), jnp.float32),
          pltpu.SemaphoreType.DMA,
      ],
  )
  def deg(dst3, zeros, ones, out, didx_all, ones_v, acc, ssem):
    c = lax.axis_index("c")
    s = lax.axis_index("s")
    pltpu.sync_copy(ones, ones_v)
    pltpu.sync_copy(dst3.at[c * NS + s], didx_all)
    pltpu.sync_copy(zeros, acc.at[pl.ds(s * per_drain, per_drain)])
    plsc.subcore_barrier()

    def body(i, carry):
      pltpu.sync_copy(ones_v, acc.at[didx_all.at[i]], add=True)
      return carry

    lax.fori_loop(0, n_outer, body, 0)
    plsc.subcore_barrier()
    pltpu.sync_copy(acc.at[pl.ds(s * per_drain, per_drain)],
                    out.at[pl.ds(c * h + s * per_drain, per_drain)])

  return deg


def _dinv(d_ref):
  return lax.rsqrt(d_ref[0, :, 0] + d_ref[1, :, 0] + 1.0)[:, None]


def _prep(x, dg3, n, r, h):
  def body(x_ref, d_ref, y_ref):
    y = x_ref[...] * _dinv(d_ref)
    y_ref[0] = y[:, :128]
    y_ref[1] = y[:, 128:]

  return pl.pallas_call(
      body,
      grid=(n // r,),
      in_specs=[pl.BlockSpec((r, 256), lambda i: (i, 0)),
                pl.BlockSpec((2, r, 128), lambda i: (0, i, 0))],
      out_specs=pl.BlockSpec((2, r, 128), lambda i: (0, i, 0)),
      out_shape=jax.ShapeDtypeStruct((2, h, 128), jnp.float32),
  )(x, dg3)


def _mid(agg3, ytab3, dg3, W1, b1, W2, n, r, h):
  hid = W1.shape[1]
  inc = W1.shape[0]

  def body(a_ref, y_ref, d_ref, w1_ref, b1_ref, w2_ref, o_ref):
    dinv = _dinv(d_ref)
    u = jnp.concatenate([a_ref[0] + y_ref[0], a_ref[1] + y_ref[1]],
                        axis=1) * dinv
    t = jnp.dot(u, w1_ref[...], preferred_element_type=jnp.float32)
    t = jnp.maximum(t + b1_ref[...], 0.0)
    y2 = jnp.dot(t, w2_ref[...], preferred_element_type=jnp.float32) * dinv
    o_ref[0] = y2[:, :128]
    o_ref[1] = y2[:, 128:]

  return pl.pallas_call(
      body,
      grid=(n // r,),
      in_specs=[pl.BlockSpec((2, r, 128), lambda i: (0, i, 0)),
                pl.BlockSpec((2, r, 128), lambda i: (0, i, 0)),
                pl.BlockSpec((2, r, 128), lambda i: (0, i, 0)),
                pl.BlockSpec((inc, hid), lambda i: (0, 0)),
                pl.BlockSpec((hid,), lambda i: (0,)),
                pl.BlockSpec((hid, 256), lambda i: (0, 0))],
      out_specs=pl.BlockSpec((2, r, 128), lambda i: (0, i, 0)),
      out_shape=jax.ShapeDtypeStruct((2, h, 128), jnp.float32),
  )(agg3, ytab3, dg3, W1, b1, W2)


def _final(aggc3, y2tab3, dg3, b2, n, r):
  outc = b2.shape[0]

  def body(a_ref, y_ref, d_ref, b2_ref, o_ref):
    o = jnp.concatenate([a_ref[0] + y_ref[0], a_ref[1] + y_ref[1]],
                        axis=1) * _dinv(d_ref)
    o_ref[...] = jnp.tanh(o + b2_ref[...])

  return pl.pallas_call(
      body,
      grid=(n // r,),
      in_specs=[pl.BlockSpec((2, r, 128), lambda i: (0, i, 0)),
                pl.BlockSpec((2, r, 128), lambda i: (0, i, 0)),
                pl.BlockSpec((2, r, 128), lambda i: (0, i, 0)),
                pl.BlockSpec((outc,), lambda i: (0,))],
      out_specs=pl.BlockSpec((r, outc), lambda i: (i, 0)),
      out_shape=jax.ShapeDtypeStruct((n, outc), jnp.float32),
  )(aggc3, y2tab3, dg3, b2)


def kernel(x, edge_index, W1, b1, W2, b2):
  n = x.shape[0]
  e = edge_index.shape[1]
  quantum = NC * NS * CH
  e_pad = ((e + quantum - 1) // quantum) * quantum
  n_quant = NS * 8
  n_pad = ((n + n_quant - 1) // n_quant) * n_quant
  r = 1000 if n % 1000 == 0 else n // 8
  h = ((n_pad + r - 1) // r) * r
  trash = n

  src = edge_index[0].astype(jnp.int32)
  dst = edge_index[1].astype(jnp.int32)
  pad = e_pad - e
  src_p = jnp.concatenate([src, jnp.zeros((pad,), jnp.int32)])
  dst_p = jnp.concatenate([dst, jnp.full((pad,), trash, jnp.int32)])
  src3 = jnp.concatenate([src_p, src_p + h]).reshape(NC * NS, -1, CH)
  dst3_agg = dst_p.reshape(NS, -1, CH)
  dst3_deg = dst_p.reshape(NC * NS, -1, CH)
  zeros128 = jnp.zeros((n_pad // NS, 128), jnp.float32)
  ones128 = jnp.ones((CH, 128), jnp.float32)

  deg_k = _make_deg(n_pad, e_pad, h)
  agg_k = _make_agg(n_pad, e_pad, h)

  dg3 = deg_k(dst3_deg, zeros128, ones128).reshape(2, h, 128)
  ytab3 = _prep(x, dg3, n, r, h)
  agg3 = agg_k(ytab3.reshape(2 * h, 128), src3, dst3_agg,
               zeros128).reshape(2, h, 128)
  y2tab3 = _mid(agg3, ytab3, dg3, W1, b1, W2, n, r, h)
  aggc3 = agg_k(y2tab3.reshape(2 * h, 128), src3, dst3_agg,
                zeros128).reshape(2, h, 128)
  return _final(aggc3, y2tab3, dg3, b2, n, r)
